# baseline (speedup 1.0000x reference)
import functools

import jax
import jax.numpy as jnp
from jax import lax
from jax.experimental import pallas as pl
from jax.experimental.pallas import tpu as pltpu
from jax.experimental.pallas import tpu_sc as plsc

F = 2
B = 16384
H = 20
D = 32
V = 1000000
N = B * H
NC = 2
NS = 16
NW = NC * NS
PER_W4 = N // 4 // NW
CHR = 256
NCH = PER_W4 // CHR
OUT_ROWS = F * N * D // 128

_mesh = plsc.VectorSubcoreMesh(core_axis_name="c", subcore_axis_name="s")


@functools.partial(
    pl.kernel,
    mesh=_mesh,
    compiler_params=pltpu.CompilerParams(use_tc_tiling_on_sc=False),
    out_type=jax.ShapeDtypeStruct((OUT_ROWS, 128), jnp.float32),
    scratch_types=[
        pltpu.VMEM((2, 4, CHR), jnp.int32),
        pltpu.VMEM((2, 4, CHR, D), jnp.float32),
        pltpu.SemaphoreType.DMA,
        pltpu.SemaphoreType.DMA,
    ],
)
def _lookup(i0, i1, i2, i3, t0_hbm, t1_hbm, out_hbm, idx_v, rows_v,
            sem_a, sem_b):
    wid = lax.axis_index("s") * NC + lax.axis_index("c")
    base = wid * PER_W4
    idxs = (i0, i1, i2, i3)
    sems = (sem_a, sem_b)

    for f, tab in enumerate((t0_hbm, t1_hbm)):
        fbase = f * (N * D // 128)

        def fire(slot, j):
            row0 = fbase + base + j * CHR
            for q in range(4):
                pltpu.sync_copy(idxs[q].at[pl.ds(row0, CHR)],
                                idx_v.at[slot, q])
                pltpu.async_copy(tab.at[idx_v.at[slot, q]],
                                 rows_v.at[slot, q], sems[slot])

        def drain_write(slot, j):
            row0 = fbase + base + j * CHR
            for q in range(4):
                pltpu.make_async_copy(out_hbm.at[pl.ds(row0, CHR),
                                                 pl.ds(0, D)],
                                      rows_v.at[slot, q], sems[slot]).wait()
                pltpu.sync_copy(rows_v.at[slot, q],
                                out_hbm.at[pl.ds(row0, CHR),
                                           pl.ds(q * 32, 32)])

        fire(0, 0)

        def body(jp, carry):
            j1 = 2 * jp + 1
            fire(1, j1)
            drain_write(0, 2 * jp)

            @pl.when(j1 + 1 < NCH)
            def _():
                fire(0, j1 + 1)

            drain_write(1, j1)
            return carry

        lax.fori_loop(0, NCH // 2, body, 0)


def kernel(indices, table_0, table_1):
    idx = indices.reshape(F * N // 4, 4).astype(jnp.int32)
    out = _lookup(idx[:, 0], idx[:, 1], idx[:, 2], idx[:, 3],
                  table_0, table_1)
    return out.reshape(F, B, H, D)
